# baseline (speedup 1.0000x reference)
import functools

import jax
import jax.numpy as jnp
from jax.experimental import pallas as pl
from jax.experimental.pallas import tpu as pltpu

B, N1, N2 = 4, 8192, 2048
C1, C2 = 64, 128
H1, H2 = 128, 128
K = 3
EPS_BN = 1e-5
BQ = 256
NB = N1 // BQ


RT = 16


def _pass_a_kernel(xyz1_ref, xyz2_ref, f2_ref, f1_ref, w1a_ref, w1b_ref,
                   b1_ref, y1_ref, s_ref, ss_ref, qr_scr, w_scr):
    b = pl.program_id(0)
    i = pl.program_id(1)

    q = xyz1_ref[0]
    r = xyz2_ref[0]

    r2 = jnp.sum(r * r, axis=1, keepdims=True).reshape(1, N2)
    qr_scr[...] = jax.lax.dot_general(q, r, (((1,), (1,)), ((), ())),
                                      preferred_element_type=jnp.float32)

    iota = jax.lax.broadcasted_iota(jnp.int32, (RT, N2), 1)

    def tile_body(t, _):
        qt = xyz1_ref[0, pl.ds(t * RT, RT), :]
        q2t = jnp.sum(qt * qt, axis=1, keepdims=True)
        qrt = qr_scr[pl.ds(t * RT, RT), :]
        dwork = jnp.maximum((q2t + r2) - 2.0 * qrt, 0.0)
        wacc = jnp.zeros((RT, N2), dtype=jnp.float32)
        wsum = jnp.zeros((RT, 1), dtype=jnp.float32)
        for _ in range(K):
            mv = jnp.min(dwork, axis=1, keepdims=True)
            cand = jnp.where(dwork <= mv, iota, N2)
            im = jnp.min(cand, axis=1, keepdims=True)
            sel = iota == im
            invd = 1.0 / jnp.maximum(jnp.sqrt(mv), 1e-8)
            wacc = wacc + jnp.where(sel, invd, 0.0)
            wsum = wsum + invd
            dwork = jnp.where(sel, jnp.inf, dwork)
        w_scr[pl.ds(t * RT, RT), :] = wacc / wsum
        return 0

    jax.lax.fori_loop(0, BQ // RT, tile_body, 0)
    interp = jnp.dot(w_scr[...], f2_ref[0], preferred_element_type=jnp.float32)

    y1 = (jnp.dot(interp, w1a_ref[...], preferred_element_type=jnp.float32)
          + jnp.dot(f1_ref[0], w1b_ref[...], preferred_element_type=jnp.float32)
          + b1_ref[...])
    y1_ref[0] = y1

    s = jnp.sum(y1, axis=0, keepdims=True)
    ss = jnp.sum(y1 * y1, axis=0, keepdims=True)

    @pl.when(jnp.logical_and(b == 0, i == 0))
    def _():
        s_ref[...] = s
        ss_ref[...] = ss

    @pl.when(jnp.logical_not(jnp.logical_and(b == 0, i == 0)))
    def _():
        s_ref[...] += s
        ss_ref[...] += ss


def _pass_b_kernel(y1_ref, sc_ref, sh_ref, w2_ref, b2_ref,
                   y2_ref, s_ref, ss_ref):
    b = pl.program_id(0)
    i = pl.program_id(1)
    h1 = jnp.maximum(y1_ref[0] * sc_ref[...] + sh_ref[...], 0.0)
    y2 = jnp.dot(h1, w2_ref[...], preferred_element_type=jnp.float32) + b2_ref[...]
    y2_ref[0] = y2
    s = jnp.sum(y2, axis=0, keepdims=True)
    ss = jnp.sum(y2 * y2, axis=0, keepdims=True)

    @pl.when(jnp.logical_and(b == 0, i == 0))
    def _():
        s_ref[...] = s
        ss_ref[...] = ss

    @pl.when(jnp.logical_not(jnp.logical_and(b == 0, i == 0)))
    def _():
        s_ref[...] += s
        ss_ref[...] += ss


def _pass_c_kernel(y2_ref, sc_ref, sh_ref, out_ref):
    h2 = jnp.maximum(y2_ref[0] * sc_ref[...] + sh_ref[...], 0.0)
    out_ref[0] = h2.T


@functools.partial(jax.jit, static_argnames=())
def kernel(xyz1, xyz2, feats1, feats2, W1, b1, g1, be1, W2, b2, g2, be2):
    f2p = jnp.swapaxes(feats2, 1, 2)
    f1p = jnp.swapaxes(feats1, 1, 2)
    w1a = W1[:, :C2].T
    w1b = W1[:, C2:].T
    w2 = W2.T
    b1r = b1.reshape(1, H1)
    b2r = b2.reshape(1, H2)

    grid = (B, NB)
    y1, s1, ss1 = pl.pallas_call(
        _pass_a_kernel,
        grid=grid,
        in_specs=[
            pl.BlockSpec((1, BQ, 3), lambda b, i: (b, i, 0)),
            pl.BlockSpec((1, N2, 3), lambda b, i: (b, 0, 0)),
            pl.BlockSpec((1, N2, C2), lambda b, i: (b, 0, 0)),
            pl.BlockSpec((1, BQ, C1), lambda b, i: (b, i, 0)),
            pl.BlockSpec((C2, H1), lambda b, i: (0, 0)),
            pl.BlockSpec((C1, H1), lambda b, i: (0, 0)),
            pl.BlockSpec((1, H1), lambda b, i: (0, 0)),
        ],
        out_specs=[
            pl.BlockSpec((1, BQ, H1), lambda b, i: (b, i, 0)),
            pl.BlockSpec((1, H1), lambda b, i: (0, 0)),
            pl.BlockSpec((1, H1), lambda b, i: (0, 0)),
        ],
        out_shape=[
            jax.ShapeDtypeStruct((B, N1, H1), jnp.float32),
            jax.ShapeDtypeStruct((1, H1), jnp.float32),
            jax.ShapeDtypeStruct((1, H1), jnp.float32),
        ],
        scratch_shapes=[
            pltpu.VMEM((BQ, N2), jnp.float32),
            pltpu.VMEM((BQ, N2), jnp.float32),
        ],
    )(xyz1, xyz2, f2p, f1p, w1a, w1b, b1r)

    cnt = float(B * N1)
    m1 = s1 / cnt
    v1 = ss1 / cnt - m1 * m1
    sc1 = g1.reshape(1, H1) / jnp.sqrt(v1 + EPS_BN)
    sh1 = be1.reshape(1, H1) - m1 * sc1

    y2, s2, ss2 = pl.pallas_call(
        _pass_b_kernel,
        grid=grid,
        in_specs=[
            pl.BlockSpec((1, BQ, H1), lambda b, i: (b, i, 0)),
            pl.BlockSpec((1, H1), lambda b, i: (0, 0)),
            pl.BlockSpec((1, H1), lambda b, i: (0, 0)),
            pl.BlockSpec((H1, H2), lambda b, i: (0, 0)),
            pl.BlockSpec((1, H2), lambda b, i: (0, 0)),
        ],
        out_specs=[
            pl.BlockSpec((1, BQ, H2), lambda b, i: (b, i, 0)),
            pl.BlockSpec((1, H2), lambda b, i: (0, 0)),
            pl.BlockSpec((1, H2), lambda b, i: (0, 0)),
        ],
        out_shape=[
            jax.ShapeDtypeStruct((B, N1, H2), jnp.float32),
            jax.ShapeDtypeStruct((1, H2), jnp.float32),
            jax.ShapeDtypeStruct((1, H2), jnp.float32),
        ],
    )(y1, sc1, sh1, w2, b2r)

    m2 = s2 / cnt
    v2 = ss2 / cnt - m2 * m2
    sc2 = g2.reshape(1, H2) / jnp.sqrt(v2 + EPS_BN)
    sh2 = be2.reshape(1, H2) - m2 * sc2

    out = pl.pallas_call(
        _pass_c_kernel,
        grid=grid,
        in_specs=[
            pl.BlockSpec((1, BQ, H2), lambda b, i: (b, i, 0)),
            pl.BlockSpec((1, H2), lambda b, i: (0, 0)),
            pl.BlockSpec((1, H2), lambda b, i: (0, 0)),
        ],
        out_specs=pl.BlockSpec((1, H2, BQ), lambda b, i: (b, 0, i)),
        out_shape=jax.ShapeDtypeStruct((B, H2, N1), jnp.float32),
    )(y2, sc2, sh2)

    return out


# speedup vs baseline: 3.5864x; 3.5864x over previous
import functools

import jax
import jax.numpy as jnp
from jax.experimental import pallas as pl
from jax.experimental.pallas import tpu as pltpu

B, N1, N2 = 4, 8192, 2048
C1, C2 = 64, 128
H1, H2 = 128, 128
K = 3
EPS_BN = 1e-5
BQ = 256
NB = N1 // BQ


def _pass_a_kernel(xyz1_ref, xyz2_ref, f2_ref, f1_ref, w1a_ref, w1b_ref,
                   b1_ref, y1_ref, s_ref, ss_ref):
    b = pl.program_id(0)
    i = pl.program_id(1)

    q = xyz1_ref[0]
    r = xyz2_ref[0]

    q2 = jnp.sum(q * q, axis=1, keepdims=True)
    r2 = jnp.sum(r * r, axis=1, keepdims=True).reshape(1, N2)
    qr = jax.lax.dot_general(q, r, (((1,), (1,)), ((), ())),
                             preferred_element_type=jnp.float32)
    d2 = jnp.maximum((q2 + r2) - 2.0 * qr, 0.0)

    iota = jax.lax.broadcasted_iota(jnp.int32, (BQ, N2), 1)
    dwork = d2
    wacc = jnp.zeros((BQ, N2), dtype=jnp.float32)
    wsum = jnp.zeros((BQ, 1), dtype=jnp.float32)
    for _ in range(K):
        mv = jnp.min(dwork, axis=1, keepdims=True)
        cand = jnp.where(dwork <= mv, iota, N2)
        im = jnp.min(cand, axis=1, keepdims=True)
        sel = iota == im
        invd = 1.0 / jnp.maximum(jnp.sqrt(mv), 1e-8)
        wacc = wacc + jnp.where(sel, invd, 0.0)
        wsum = wsum + invd
        dwork = jnp.where(sel, jnp.inf, dwork)

    wacc = wacc / wsum
    interp = jnp.dot(wacc, f2_ref[0], preferred_element_type=jnp.float32)

    y1 = (jnp.dot(interp, w1a_ref[...], preferred_element_type=jnp.float32)
          + jnp.dot(f1_ref[0], w1b_ref[...], preferred_element_type=jnp.float32)
          + b1_ref[...])
    y1_ref[0] = y1

    s = jnp.sum(y1, axis=0, keepdims=True)
    ss = jnp.sum(y1 * y1, axis=0, keepdims=True)

    @pl.when(jnp.logical_and(b == 0, i == 0))
    def _():
        s_ref[...] = s
        ss_ref[...] = ss

    @pl.when(jnp.logical_not(jnp.logical_and(b == 0, i == 0)))
    def _():
        s_ref[...] += s
        ss_ref[...] += ss


def _pass_b_kernel(y1_ref, sc_ref, sh_ref, w2_ref, b2_ref,
                   y2_ref, s_ref, ss_ref):
    b = pl.program_id(0)
    i = pl.program_id(1)
    h1 = jnp.maximum(y1_ref[0] * sc_ref[...] + sh_ref[...], 0.0)
    y2 = jnp.dot(h1, w2_ref[...], preferred_element_type=jnp.float32) + b2_ref[...]
    y2_ref[0] = y2
    s = jnp.sum(y2, axis=0, keepdims=True)
    ss = jnp.sum(y2 * y2, axis=0, keepdims=True)

    @pl.when(jnp.logical_and(b == 0, i == 0))
    def _():
        s_ref[...] = s
        ss_ref[...] = ss

    @pl.when(jnp.logical_not(jnp.logical_and(b == 0, i == 0)))
    def _():
        s_ref[...] += s
        ss_ref[...] += ss


def _pass_c_kernel(y2_ref, sc_ref, sh_ref, out_ref):
    h2 = jnp.maximum(y2_ref[0] * sc_ref[...] + sh_ref[...], 0.0)
    out_ref[0] = h2.T


@functools.partial(jax.jit, static_argnames=())
def kernel(xyz1, xyz2, feats1, feats2, W1, b1, g1, be1, W2, b2, g2, be2):
    f2p = jnp.swapaxes(feats2, 1, 2)
    f1p = jnp.swapaxes(feats1, 1, 2)
    w1a = W1[:, :C2].T
    w1b = W1[:, C2:].T
    w2 = W2.T
    b1r = b1.reshape(1, H1)
    b2r = b2.reshape(1, H2)

    grid = (B, NB)
    y1, s1, ss1 = pl.pallas_call(
        _pass_a_kernel,
        grid=grid,
        in_specs=[
            pl.BlockSpec((1, BQ, 3), lambda b, i: (b, i, 0)),
            pl.BlockSpec((1, N2, 3), lambda b, i: (b, 0, 0)),
            pl.BlockSpec((1, N2, C2), lambda b, i: (b, 0, 0)),
            pl.BlockSpec((1, BQ, C1), lambda b, i: (b, i, 0)),
            pl.BlockSpec((C2, H1), lambda b, i: (0, 0)),
            pl.BlockSpec((C1, H1), lambda b, i: (0, 0)),
            pl.BlockSpec((1, H1), lambda b, i: (0, 0)),
        ],
        out_specs=[
            pl.BlockSpec((1, BQ, H1), lambda b, i: (b, i, 0)),
            pl.BlockSpec((1, H1), lambda b, i: (0, 0)),
            pl.BlockSpec((1, H1), lambda b, i: (0, 0)),
        ],
        out_shape=[
            jax.ShapeDtypeStruct((B, N1, H1), jnp.float32),
            jax.ShapeDtypeStruct((1, H1), jnp.float32),
            jax.ShapeDtypeStruct((1, H1), jnp.float32),
        ],
    )(xyz1, xyz2, f2p, f1p, w1a, w1b, b1r)

    cnt = float(B * N1)
    m1 = s1 / cnt
    v1 = ss1 / cnt - m1 * m1
    sc1 = g1.reshape(1, H1) / jnp.sqrt(v1 + EPS_BN)
    sh1 = be1.reshape(1, H1) - m1 * sc1

    y2, s2, ss2 = pl.pallas_call(
        _pass_b_kernel,
        grid=grid,
        in_specs=[
            pl.BlockSpec((1, BQ, H1), lambda b, i: (b, i, 0)),
            pl.BlockSpec((1, H1), lambda b, i: (0, 0)),
            pl.BlockSpec((1, H1), lambda b, i: (0, 0)),
            pl.BlockSpec((H1, H2), lambda b, i: (0, 0)),
            pl.BlockSpec((1, H2), lambda b, i: (0, 0)),
        ],
        out_specs=[
            pl.BlockSpec((1, BQ, H2), lambda b, i: (b, i, 0)),
            pl.BlockSpec((1, H2), lambda b, i: (0, 0)),
            pl.BlockSpec((1, H2), lambda b, i: (0, 0)),
        ],
        out_shape=[
            jax.ShapeDtypeStruct((B, N1, H2), jnp.float32),
            jax.ShapeDtypeStruct((1, H2), jnp.float32),
            jax.ShapeDtypeStruct((1, H2), jnp.float32),
        ],
    )(y1, sc1, sh1, w2, b2r)

    m2 = s2 / cnt
    v2 = ss2 / cnt - m2 * m2
    sc2 = g2.reshape(1, H2) / jnp.sqrt(v2 + EPS_BN)
    sh2 = be2.reshape(1, H2) - m2 * sc2

    out = pl.pallas_call(
        _pass_c_kernel,
        grid=grid,
        in_specs=[
            pl.BlockSpec((1, BQ, H2), lambda b, i: (b, i, 0)),
            pl.BlockSpec((1, H2), lambda b, i: (0, 0)),
            pl.BlockSpec((1, H2), lambda b, i: (0, 0)),
        ],
        out_specs=pl.BlockSpec((1, H2, BQ), lambda b, i: (b, 0, i)),
        out_shape=jax.ShapeDtypeStruct((B, H2, N1), jnp.float32),
    )(y2, sc2, sh2)

    return out
